# baseline (speedup 1.0000x reference)
import functools

import jax
import jax.numpy as jnp
from jax import lax
from jax.experimental import pallas as pl
from jax.experimental.pallas import tpu as pltpu
from jax.experimental.pallas import tpu_sc as plsc

N = 10000
E = 320000
D = 128
H = 128
G = 256
C = 10
NP = 10240
NACC = 10016
NW = 32
EW_A = E // NW
SEGCAP = 10752

_mesh = plsc.VectorSubcoreMesh(core_axis_name="c", subcore_axis_name="s")


def _sc_prep_body(col_hbm, row_hbm, ew_hbm, degp_hbm, colc_hbm, rowc_hbm,
                  cnt_hbm,
                  colv, rowv, ewv, colcb, rowcb, rowchb, cntv,
                  deg0, deg1, deg2):
    cid = lax.axis_index("c")
    sid = lax.axis_index("s")
    wid = sid * 2 + cid
    base = wid * EW_A
    degs = [deg0, deg1, deg2]
    lane = lax.iota(jnp.int32, 16)

    pltpu.sync_copy(row_hbm.at[pl.ds(base, EW_A)], rowv)
    pltpu.sync_copy(col_hbm.at[pl.ds(base, EW_A)], colv)

    def zero(j, _):
        z = jnp.zeros((16,), jnp.float32)
        deg0[pl.ds(j * 16, 16)] = z
        deg1[pl.ds(j * 16, 16)] = z
        deg2[pl.ds(j * 16, 16)] = z
        return 0
    lax.fori_loop(0, NP // 16, zero, 0)
    cntv[pl.ds(0, 16)] = jnp.zeros((16,), jnp.int32)

    for b in range(3):
        pltpu.sync_copy(ew_hbm.at[pl.ds(b * E + base, EW_A)], ewv)

        def pad(j, _):
            sl = pl.ds(j * 16, 16)
            colcb[sl] = N + lane
            rowcb[sl] = jnp.zeros((16,), jnp.int32)
            rowchb[sl] = jnp.zeros((16,), jnp.int32)
            return 0
        lax.fori_loop(0, SEGCAP // 16, pad, 0)

        def step(j, off, b=b):
            sl = pl.ds(j * 16, 16)
            cv = colv[sl]
            rv = rowv[sl]
            ev = ewv[sl]
            m = ev > 0.5
            plsc.addupdate_scatter(degs[b], [cv], ev)
            cs = plsc.cumsum(m.astype(jnp.int32))
            pos = jnp.where(m, off + cs - 1, 0)
            plsc.store_scatter(colcb, [pos], cv, mask=m)
            plsc.store_scatter(rowcb, [pos], rv, mask=m)
            plsc.store_scatter(rowchb, [pos], rv + N, mask=m)
            return off + jnp.max(cs)
        cnt_b = lax.fori_loop(0, EW_A // 16, step, jnp.int32(0))
        cntv[pl.ds(0, 16)] = jnp.where(lane == b, cnt_b, cntv[pl.ds(0, 16)])

        pltpu.sync_copy(colcb, colc_hbm.at[pl.ds((b * NW + wid) * SEGCAP,
                                                 SEGCAP)])
        pltpu.sync_copy(rowcb,
                        rowc_hbm.at[pl.ds(((b * 2 + 0) * NW + wid) * SEGCAP,
                                          SEGCAP)])
        pltpu.sync_copy(rowchb,
                        rowc_hbm.at[pl.ds(((b * 2 + 1) * NW + wid) * SEGCAP,
                                          SEGCAP)])
        pltpu.sync_copy(degs[b], degp_hbm.at[pl.ds((wid * 3 + b) * NP, NP)])

    pltpu.sync_copy(cntv.at[pl.ds(0, 8)], cnt_hbm.at[pl.ds(wid * 8, 8)])


_sc_prep = pl.kernel(
    _sc_prep_body,
    out_type=(
        jax.ShapeDtypeStruct((NW * 3 * NP,), jnp.float32),
        jax.ShapeDtypeStruct((3 * NW * SEGCAP,), jnp.int32),
        jax.ShapeDtypeStruct((6 * NW * SEGCAP,), jnp.int32),
        jax.ShapeDtypeStruct((NW * 8,), jnp.int32),
    ),
    mesh=_mesh,
    scratch_types=[
        pltpu.VMEM((EW_A,), jnp.int32),
        pltpu.VMEM((EW_A,), jnp.int32),
        pltpu.VMEM((EW_A,), jnp.float32),
        pltpu.VMEM((SEGCAP,), jnp.int32),
        pltpu.VMEM((SEGCAP,), jnp.int32),
        pltpu.VMEM((SEGCAP,), jnp.int32),
        pltpu.VMEM((16,), jnp.int32),
        pltpu.VMEM((NP,), jnp.float32),
        pltpu.VMEM((NP,), jnp.float32),
        pltpu.VMEM((NP,), jnp.float32),
    ],
    compiler_params=pltpu.CompilerParams(needs_layout_passes=False),
)


def _sc_agg_body(hb0, hb1, hb2, rowc_hbm, colc_hbm, cnt_hbm, s0, s1, s2,
                 ab0, ab1, cb0, cb1, cb2,
                 m0, m1, cntbuf, zbuf,
                 acc0,
                 si0, si1, sg0, sg1, ss0, ss1):
    cid = lax.axis_index("c")
    sid = lax.axis_index("s")
    hbs = [hb0, hb1, hb2]
    outs = [s0, s1, s2]
    abuf = [ab0, ab1]
    cbuf = [cb0, cb1, cb2]
    msg = [m0, m1]
    sem_i = [si0, si1]
    sem_g = [sg0, sg1]
    sem_s = [ss0, ss1]
    lane = lax.iota(jnp.int32, 16)

    zbase = sid * 626

    def zrow(j, _):
        z = jnp.zeros((16,), jnp.float32)
        for k in range(4):
            zbuf[j, pl.ds(k * 16, 16)] = z
        return 0
    lax.fori_loop(0, 128, zrow, 0)
    cntbuf[pl.ds(0, 16)] = jnp.zeros((16,), jnp.int32)

    def zero_acc():
        for k in range(4):
            pltpu.sync_copy(zbuf, acc0.at[pl.ds(zbase + k * 128, 128)])
        pltpu.sync_copy(zbuf.at[pl.ds(0, 114)],
                        acc0.at[pl.ds(zbase + 512, 114)])

    def pipeline(b, goff, coff, gsteady):
        def idx_start(c, p, r):
            pltpu.async_copy(rowc_hbm.at[pl.ds(goff + c * 128, 128)],
                             abuf[p], sem_i[p])
            pltpu.async_copy(colc_hbm.at[pl.ds(coff + c * 128, 128)],
                             cbuf[r], sem_i[p])

        def idx_wait(c, p, r):
            pltpu.make_async_copy(rowc_hbm.at[pl.ds(goff + c * 128, 128)],
                                  abuf[p], sem_i[p]).wait()
            pltpu.make_async_copy(colc_hbm.at[pl.ds(coff + c * 128, 128)],
                                  cbuf[r], sem_i[p]).wait()

        def gat_start(q):
            pltpu.async_copy(hbs[b].at[abuf[q]], msg[q], sem_g[q])

        def gat_wait(q):
            pltpu.make_async_copy(hbs[b].at[abuf[q]], msg[q], sem_g[q]).wait()

        def sct_start(p, r):
            pltpu.async_copy(msg[p], acc0.at[cbuf[r]], sem_s[p], add=True)

        def sct_wait(p, r):
            pltpu.make_async_copy(msg[p], acc0.at[cbuf[r]], sem_s[p]).wait()

        def full_body(c, p, q, r):
            sct_wait(q, r)
            idx_wait(c - 1, q, (r + 2) % 3)
            gat_start(q)
            gat_wait(p)
            sct_start(p, (r + 1) % 3)
            idx_start(c, p, r)

        idx_start(0, 0, 0)
        idx_wait(0, 0, 0)
        gat_start(0)
        idx_start(1, 1, 1)
        idx_wait(1, 1, 1)
        gat_start(1)
        gat_wait(0)
        sct_start(0, 0)
        idx_start(2, 0, 2)

        def steady(g, _):
            c = 3 + 6 * g
            for o in range(6):
                full_body(c + o, (3 + o) % 2, (o + 2) % 2, o % 3)
            return 0
        lax.fori_loop(0, gsteady, steady, 0)
        nc = 6 * gsteady + 6
        full_body(nc - 3, 1, 0, 0)
        full_body(nc - 2, 0, 1, 1)
        full_body(nc - 1, 1, 0, 2)

        sct_wait(1, 0)
        idx_wait(nc - 1, 1, 2)
        gat_start(1)
        gat_wait(0)
        sct_start(0, 1)
        gat_wait(1)
        sct_start(1, 2)
        sct_wait(0, 1)
        sct_wait(1, 2)

    for b in range(3):
        zero_acc()
        plsc.subcore_barrier()
        for wi in range(2):
            w = 2 * sid + wi
            pltpu.sync_copy(cnt_hbm.at[pl.ds(w * 8, 8)],
                            cntbuf.at[pl.ds(0, 8)])
            cv = cntbuf[pl.ds(0, 16)]
            cnt_b = jnp.minimum(jnp.max(jnp.where(lane == b, cv, 0)), EW_A)
            nchunks = (cnt_b + 127) // 128
            gsteady = jnp.maximum((nchunks + 5) // 6, 1) - 1
            goff = ((b * 2 + cid) * NW + w) * SEGCAP
            coff = (b * NW + w) * SEGCAP
            pipeline(b, goff, coff, gsteady)
        plsc.subcore_barrier()
        pltpu.sync_copy(acc0.at[pl.ds(zbase, 626)],
                        outs[b].at[cid, pl.ds(zbase, 626)])


_sc_agg = pl.kernel(
    _sc_agg_body,
    out_type=(
        jax.ShapeDtypeStruct((2, NACC, 64), jnp.float32),
        jax.ShapeDtypeStruct((2, NACC, 64), jnp.float32),
        jax.ShapeDtypeStruct((2, NACC, 64), jnp.float32),
    ),
    mesh=_mesh,
    scratch_types=(
        [pltpu.VMEM((128,), jnp.int32) for _ in range(5)]
        + [pltpu.VMEM((128, 64), jnp.float32) for _ in range(2)]
        + [pltpu.VMEM((16,), jnp.int32),
           pltpu.VMEM((128, 64), jnp.float32)]
        + [pltpu.VMEM_SHARED((NACC, 64), jnp.float32)]
        + [pltpu.SemaphoreType.DMA for _ in range(6)]
    ),
    compiler_params=pltpu.CompilerParams(use_tc_tiling_on_sc=False,
                                         needs_layout_passes=False),
)


def _dinv_body(degp_ref, dinv_ref):
    deg = jnp.sum(degp_ref[...], axis=0)
    dinv_ref[...] = jnp.where(deg > 0, lax.rsqrt(jnp.maximum(deg, 1e-12)), 0.0)


def _tc_dinv(degp):
    return pl.pallas_call(
        _dinv_body,
        out_shape=jax.ShapeDtypeStruct((3, NP), jnp.float32),
    )(degp)


def _scale_body(x_ref, dinv_ref, h0_ref, h1_ref, h2_ref):
    x = x_ref[...]
    outs = [h0_ref, h1_ref, h2_ref]
    for b in range(3):
        db = dinv_ref[:, b][:, None]
        outs[b][0] = db * x[:, :64]
        outs[b][1] = db * x[:, 64:]


def _tc_scale(x, dinv):
    blk = 1000
    grid = N // blk
    out_specs = tuple(
        pl.BlockSpec((2, blk, 64), lambda i: (0, i, 0)) for _ in range(3))
    return pl.pallas_call(
        _scale_body,
        grid=(grid,),
        in_specs=[
            pl.BlockSpec((blk, D), lambda i: (i, 0)),
            pl.BlockSpec((blk, 3), lambda i: (i, 0)),
        ],
        out_specs=out_specs,
        out_shape=tuple(
            jax.ShapeDtypeStruct((2, N, 64), jnp.float32) for _ in range(3)),
    )(x, dinv)


def _layer_a_body(h_ref, s0_ref, s1_ref, s2_ref, dinv_ref,
                  w_ref, bias_ref, u_ref, stats_ref):
    srefs = [s0_ref, s1_ref, s2_ref]
    acc = None
    for b in range(3):
        sb = jnp.concatenate([srefs[b][0], srefs[b][1]], axis=1)
        pre = dinv_ref[:, b][:, None] * sb
        ob = jnp.maximum(
            jnp.dot(pre, w_ref[b], preferred_element_type=jnp.float32)
            + bias_ref[b, :][None, :], 0.0)
        acc = ob if acc is None else acc + ob
    xi = jnp.maximum(
        jnp.dot(h_ref[...], w_ref[3], preferred_element_type=jnp.float32)
        + bias_ref[3, :][None, :], 0.0)
    u = acc + xi
    u_ref[...] = u

    @pl.when(pl.program_id(0) == 0)
    def _():
        stats_ref[...] = jnp.zeros_like(stats_ref)

    stats_ref[0:1, :] += jnp.sum(u, axis=0, keepdims=True)
    stats_ref[1:2, :] += jnp.sum(u * u, axis=0, keepdims=True)


def _tc_layer_a(h, s0, s1, s2, dinv, w4, b4):
    blk = 1000
    grid = N // blk
    sspec = pl.BlockSpec((2, blk, 64), lambda i: (0, i, 0))
    return pl.pallas_call(
        _layer_a_body,
        grid=(grid,),
        in_specs=[
            pl.BlockSpec((blk, D), lambda i: (i, 0)),
            sspec, sspec, sspec,
            pl.BlockSpec((blk, 3), lambda i: (i, 0)),
            pl.BlockSpec((4, D, H), lambda i: (0, 0, 0)),
            pl.BlockSpec((4, H), lambda i: (0, 0)),
        ],
        out_specs=(
            pl.BlockSpec((blk, H), lambda i: (i, 0)),
            pl.BlockSpec((2, H), lambda i: (0, 0)),
        ),
        out_shape=(
            jax.ShapeDtypeStruct((N, H), jnp.float32),
            jax.ShapeDtypeStruct((2, H), jnp.float32),
        ),
    )(h, s0, s1, s2, dinv, w4, b4)


def _layer_b_body(u_ref, stats_ref, gb_ref, dinv_ref, h_ref,
                  h0_ref, h1_ref, h2_ref, *, make_hb):
    mean = stats_ref[0:1, :] / N
    var = stats_ref[1:2, :] / N - mean * mean
    rstd = lax.rsqrt(var + 1e-5)
    hn = (u_ref[...] - mean) * rstd * gb_ref[0:1, :] + gb_ref[1:2, :]
    h_ref[...] = hn
    if make_hb:
        outs = [h0_ref, h1_ref, h2_ref]
        for b in range(3):
            db = dinv_ref[:, b][:, None]
            outs[b][0] = db * hn[:, :64]
            outs[b][1] = db * hn[:, 64:]


def _tc_layer_b(u, stats, gb, dinv, make_hb):
    blk = 1000
    grid = N // blk
    body = functools.partial(_layer_b_body, make_hb=make_hb)
    if make_hb:
        out_specs = (
            pl.BlockSpec((blk, H), lambda i: (i, 0)),
            pl.BlockSpec((2, blk, 64), lambda i: (0, i, 0)),
            pl.BlockSpec((2, blk, 64), lambda i: (0, i, 0)),
            pl.BlockSpec((2, blk, 64), lambda i: (0, i, 0)),
        )
        out_shape = (
            jax.ShapeDtypeStruct((N, H), jnp.float32),
            jax.ShapeDtypeStruct((2, N, 64), jnp.float32),
            jax.ShapeDtypeStruct((2, N, 64), jnp.float32),
            jax.ShapeDtypeStruct((2, N, 64), jnp.float32),
        )
    else:
        def body(u_ref, stats_ref, gb_ref, dinv_ref, h_ref):
            _layer_b_body(u_ref, stats_ref, gb_ref, dinv_ref, h_ref,
                          None, None, None, make_hb=False)
        out_specs = pl.BlockSpec((blk, H), lambda i: (i, 0))
        out_shape = jax.ShapeDtypeStruct((N, H), jnp.float32)
    return pl.pallas_call(
        body,
        grid=(grid,),
        in_specs=[
            pl.BlockSpec((blk, H), lambda i: (i, 0)),
            pl.BlockSpec((2, H), lambda i: (0, 0)),
            pl.BlockSpec((2, H), lambda i: (0, 0)),
            pl.BlockSpec((blk, 3), lambda i: (i, 0)),
        ],
        out_specs=out_specs,
        out_shape=out_shape,
    )(u, stats, gb, dinv)


def _pool_body(h_ref, batch_ref, pooled_ref, counts_ref):
    bt = batch_ref[0, 0, :]
    gi = lax.broadcasted_iota(jnp.int32, (G, bt.shape[0]), 0)
    oh = (gi == bt[None, :]).astype(jnp.float32)

    @pl.when(pl.program_id(0) == 0)
    def _():
        pooled_ref[...] = jnp.zeros_like(pooled_ref)
        counts_ref[...] = jnp.zeros_like(counts_ref)

    pooled_ref[...] += jnp.dot(oh, h_ref[...],
                               preferred_element_type=jnp.float32)
    counts_ref[...] += jnp.sum(oh, axis=1)[None, :]


def _tc_pool(h, batch3d):
    blk = 1000
    grid = N // blk
    return pl.pallas_call(
        _pool_body,
        grid=(grid,),
        in_specs=[
            pl.BlockSpec((blk, H), lambda i: (i, 0)),
            pl.BlockSpec((1, 1, blk), lambda i: (i, 0, 0)),
        ],
        out_specs=(
            pl.BlockSpec((G, H), lambda i: (0, 0)),
            pl.BlockSpec((1, G), lambda i: (0, 0)),
        ),
        out_shape=(
            jax.ShapeDtypeStruct((G, H), jnp.float32),
            jax.ShapeDtypeStruct((1, G), jnp.float32),
        ),
    )(h, batch3d)


def _mlp_body(pooled_ref, counts_ref, w1a_ref, w1b_ref, b1_ref,
              w2_ref, b2_ref, out_ref):
    cnt = counts_ref[0, :][:, None] / 40.0
    z = (jnp.dot(pooled_ref[...], w1a_ref[...],
                 preferred_element_type=jnp.float32)
         + cnt * w1b_ref[0:1, :] + b1_ref[0:1, :])
    z = jnp.maximum(z, 0.0)
    out_ref[...] = (jnp.dot(z, w2_ref[...],
                            preferred_element_type=jnp.float32)
                    + b2_ref[0:1, :])


def _tc_mlp(pooled, counts, w1a, w1b, b1, w2, b2):
    return pl.pallas_call(
        _mlp_body,
        out_shape=jax.ShapeDtypeStruct((G, C), jnp.float32),
    )(pooled, counts, w1a, w1b, b1, w2, b2)


def kernel(x, edge_attr, params, edge_index, batch):
    row = edge_index[0]
    col = edge_index[1]
    ewT = jnp.transpose(edge_attr[:, :3]).reshape(-1)

    degp, colc, rowc, cnt = _sc_prep(col, row, ewT)
    dinv_p = _tc_dinv(degp.reshape(NW, 3, NP))
    dinv = jnp.transpose(dinv_p)

    hb = _tc_scale(x, dinv)
    hb = [a.reshape(2 * N, 64) for a in hb]

    h = x
    for li, lyr in enumerate(params["layers"]):
        s0, s1, s2 = _sc_agg(hb[0], hb[1], hb[2], rowc, colc, cnt)
        w4 = jnp.stack([lyr["Ws"], lyr["Wd"], lyr["Wt"], lyr["Wi"]])
        b4 = jnp.stack([lyr["bs"], lyr["bd"], lyr["bt"], lyr["bi"]])
        u, stats = _tc_layer_a(h, s0, s1, s2, dinv, w4, b4)
        gb = jnp.stack([lyr["g"], lyr["be"]])
        if li == 0:
            h, h0, h1, h2 = _tc_layer_b(u, stats, gb, dinv, True)
            hb = [h0.reshape(2 * N, 64), h1.reshape(2 * N, 64),
                  h2.reshape(2 * N, 64)]
        else:
            h = _tc_layer_b(u, stats, gb, dinv, False)

    batch3d = batch.reshape(10, 1, N // 10)
    pooled, counts = _tc_pool(h, batch3d)
    w1a = params["fc1_W"][:H, :]
    w1b = params["fc1_W"][H:, :]
    return _tc_mlp(pooled, counts, w1a, w1b,
                   params["fc1_b"][None, :], params["fc2_W"],
                   params["fc2_b"][None, :])


# speedup vs baseline: 1.0003x; 1.0003x over previous
import functools

import jax
import jax.numpy as jnp
from jax import lax
from jax.experimental import pallas as pl
from jax.experimental.pallas import tpu as pltpu
from jax.experimental.pallas import tpu_sc as plsc

N = 10000
E = 320000
D = 128
H = 128
G = 256
C = 10
NP = 10240
NACC = 10016
NW = 32
EW_A = E // NW
SEGCAP = 10752

_mesh = plsc.VectorSubcoreMesh(core_axis_name="c", subcore_axis_name="s")


def _sc_prep_body(col_hbm, row_hbm, ew_hbm, degp_hbm, colc_hbm, rowc_hbm,
                  cnt_hbm,
                  colv, rowv, ewv, colcb, rowcb, rowchb, cntv,
                  deg0, deg1, deg2):
    cid = lax.axis_index("c")
    sid = lax.axis_index("s")
    wid = sid * 2 + cid
    base = wid * EW_A
    degs = [deg0, deg1, deg2]
    lane = lax.iota(jnp.int32, 16)

    pltpu.sync_copy(row_hbm.at[pl.ds(base, EW_A)], rowv)
    pltpu.sync_copy(col_hbm.at[pl.ds(base, EW_A)], colv)

    def zero(j, _):
        z = jnp.zeros((16,), jnp.float32)
        deg0[pl.ds(j * 16, 16)] = z
        deg1[pl.ds(j * 16, 16)] = z
        deg2[pl.ds(j * 16, 16)] = z
        return 0
    lax.fori_loop(0, NP // 16, zero, 0)
    cntv[pl.ds(0, 16)] = jnp.zeros((16,), jnp.int32)

    for b in range(3):
        pltpu.sync_copy(ew_hbm.at[pl.ds(b * E + base, EW_A)], ewv)

        def pad(j, _):
            sl = pl.ds(j * 16, 16)
            colcb[sl] = N + lane
            rowcb[sl] = jnp.zeros((16,), jnp.int32)
            rowchb[sl] = jnp.zeros((16,), jnp.int32)
            return 0
        lax.fori_loop(0, SEGCAP // 16, pad, 0)

        def step(j, off, b=b):
            sl = pl.ds(j * 16, 16)
            cv = colv[sl]
            rv = rowv[sl]
            ev = ewv[sl]
            m = ev > 0.5
            plsc.addupdate_scatter(degs[b], [cv], ev)
            cs = plsc.cumsum(m.astype(jnp.int32))
            pos = jnp.where(m, off + cs - 1, 0)
            plsc.store_scatter(colcb, [pos], cv, mask=m)
            plsc.store_scatter(rowcb, [pos], rv + (2 * b) * N, mask=m)
            plsc.store_scatter(rowchb, [pos], rv + (2 * b + 1) * N, mask=m)
            return off + jnp.max(cs)
        cnt_b = lax.fori_loop(0, EW_A // 16, step, jnp.int32(0))
        cntv[pl.ds(0, 16)] = jnp.where(lane == b, cnt_b, cntv[pl.ds(0, 16)])

        pltpu.sync_copy(colcb, colc_hbm.at[pl.ds((b * NW + wid) * SEGCAP,
                                                 SEGCAP)])
        pltpu.sync_copy(rowcb,
                        rowc_hbm.at[pl.ds(((b * 2 + 0) * NW + wid) * SEGCAP,
                                          SEGCAP)])
        pltpu.sync_copy(rowchb,
                        rowc_hbm.at[pl.ds(((b * 2 + 1) * NW + wid) * SEGCAP,
                                          SEGCAP)])
        pltpu.sync_copy(degs[b], degp_hbm.at[pl.ds((wid * 3 + b) * NP, NP)])

    pltpu.sync_copy(cntv.at[pl.ds(0, 8)], cnt_hbm.at[pl.ds(wid * 8, 8)])


_sc_prep = pl.kernel(
    _sc_prep_body,
    out_type=(
        jax.ShapeDtypeStruct((NW * 3 * NP,), jnp.float32),
        jax.ShapeDtypeStruct((3 * NW * SEGCAP,), jnp.int32),
        jax.ShapeDtypeStruct((6 * NW * SEGCAP,), jnp.int32),
        jax.ShapeDtypeStruct((NW * 8,), jnp.int32),
    ),
    mesh=_mesh,
    scratch_types=[
        pltpu.VMEM((EW_A,), jnp.int32),
        pltpu.VMEM((EW_A,), jnp.int32),
        pltpu.VMEM((EW_A,), jnp.float32),
        pltpu.VMEM((SEGCAP,), jnp.int32),
        pltpu.VMEM((SEGCAP,), jnp.int32),
        pltpu.VMEM((SEGCAP,), jnp.int32),
        pltpu.VMEM((16,), jnp.int32),
        pltpu.VMEM((NP,), jnp.float32),
        pltpu.VMEM((NP,), jnp.float32),
        pltpu.VMEM((NP,), jnp.float32),
    ],
    compiler_params=pltpu.CompilerParams(needs_layout_passes=False),
)


def _sc_agg_body(hbcat, rowc_hbm, colc_hbm, cnt_hbm, s0, s1, s2,
                 ab0, ab1, cb0, cb1, cb2,
                 m0, m1, cntbuf, zbuf,
                 acc0,
                 si0, si1, sg0, sg1, ss0, ss1):
    cid = lax.axis_index("c")
    sid = lax.axis_index("s")
    outs = [s0, s1, s2]
    abuf = [ab0, ab1]
    cbuf = [cb0, cb1, cb2]
    msg = [m0, m1]
    sem_i = [si0, si1]
    sem_g = [sg0, sg1]
    sem_s = [ss0, ss1]
    lane = lax.iota(jnp.int32, 16)

    zbase = sid * 626

    def zrow(j, _):
        z = jnp.zeros((16,), jnp.float32)
        for k in range(4):
            zbuf[j, pl.ds(k * 16, 16)] = z
        return 0
    lax.fori_loop(0, 128, zrow, 0)
    cntbuf[pl.ds(0, 16)] = jnp.zeros((16,), jnp.int32)

    def zero_acc():
        for k in range(4):
            pltpu.sync_copy(zbuf, acc0.at[pl.ds(zbase + k * 128, 128)])
        pltpu.sync_copy(zbuf.at[pl.ds(0, 114)],
                        acc0.at[pl.ds(zbase + 512, 114)])

    def pipeline(nc0, goff0, goff1, coff0, coff1, gsteady):
        def gsl(c):
            return jnp.where(c < nc0, goff0 + c * 128,
                             goff1 + (c - nc0) * 128)

        def csl(c):
            return jnp.where(c < nc0, coff0 + c * 128,
                             coff1 + (c - nc0) * 128)

        def idx_start(c, p, r):
            pltpu.async_copy(rowc_hbm.at[pl.ds(gsl(c), 128)],
                             abuf[p], sem_i[p])
            pltpu.async_copy(colc_hbm.at[pl.ds(csl(c), 128)],
                             cbuf[r], sem_i[p])

        def idx_wait(c, p, r):
            pltpu.make_async_copy(rowc_hbm.at[pl.ds(gsl(c), 128)],
                                  abuf[p], sem_i[p]).wait()
            pltpu.make_async_copy(colc_hbm.at[pl.ds(csl(c), 128)],
                                  cbuf[r], sem_i[p]).wait()

        def gat_start(q):
            pltpu.async_copy(hbcat.at[abuf[q]], msg[q], sem_g[q])

        def gat_wait(q):
            pltpu.make_async_copy(hbcat.at[abuf[q]], msg[q], sem_g[q]).wait()

        def sct_start(p, r):
            pltpu.async_copy(msg[p], acc0.at[cbuf[r]], sem_s[p], add=True)

        def sct_wait(p, r):
            pltpu.make_async_copy(msg[p], acc0.at[cbuf[r]], sem_s[p]).wait()

        def full_body(c, p, q, r):
            sct_wait(q, r)
            idx_wait(c - 1, q, (r + 2) % 3)
            gat_start(q)
            gat_wait(p)
            sct_start(p, (r + 1) % 3)
            idx_start(c, p, r)

        idx_start(0, 0, 0)
        idx_wait(0, 0, 0)
        gat_start(0)
        idx_start(1, 1, 1)
        idx_wait(1, 1, 1)
        gat_start(1)
        gat_wait(0)
        sct_start(0, 0)
        idx_start(2, 0, 2)

        def steady(g, _):
            c = 3 + 6 * g
            for o in range(6):
                full_body(c + o, (3 + o) % 2, (o + 2) % 2, o % 3)
            return 0
        lax.fori_loop(0, gsteady, steady, 0)
        nc = 6 * gsteady + 6
        full_body(nc - 3, 1, 0, 0)
        full_body(nc - 2, 0, 1, 1)
        full_body(nc - 1, 1, 0, 2)

        sct_wait(1, 0)
        idx_wait(nc - 1, 1, 2)
        gat_start(1)
        gat_wait(0)
        sct_start(0, 1)
        gat_wait(1)
        sct_start(1, 2)
        sct_wait(0, 1)
        sct_wait(1, 2)

    w0 = 2 * sid
    pltpu.sync_copy(cnt_hbm.at[pl.ds(w0 * 8, 8)], cntbuf.at[pl.ds(0, 8)])
    pltpu.sync_copy(cnt_hbm.at[pl.ds(w0 * 8 + 8, 8)], cntbuf.at[pl.ds(8, 8)])
    cv = cntbuf[pl.ds(0, 16)]
    for b in range(3):
        zero_acc()
        plsc.subcore_barrier()
        c0 = jnp.minimum(jnp.max(jnp.where(lane == b, cv, 0)), EW_A)
        c1 = jnp.minimum(jnp.max(jnp.where(lane == 8 + b, cv, 0)), EW_A)
        nc0 = jnp.maximum(6 * (((c0 + 127) // 128 + 5) // 6), 6)
        nc1 = jnp.maximum(6 * (((c1 + 127) // 128 + 5) // 6), 6)
        gsteady = (nc0 + nc1) // 6 - 1
        goff0 = ((b * 2 + cid) * NW + w0) * SEGCAP
        goff1 = goff0 + SEGCAP
        coff0 = (b * NW + w0) * SEGCAP
        coff1 = coff0 + SEGCAP
        pipeline(nc0, goff0, goff1, coff0, coff1, gsteady)
        plsc.subcore_barrier()
        pltpu.sync_copy(acc0.at[pl.ds(zbase, 626)],
                        outs[b].at[cid, pl.ds(zbase, 626)])


_sc_agg = pl.kernel(
    _sc_agg_body,
    out_type=(
        jax.ShapeDtypeStruct((2, NACC, 64), jnp.float32),
        jax.ShapeDtypeStruct((2, NACC, 64), jnp.float32),
        jax.ShapeDtypeStruct((2, NACC, 64), jnp.float32),
    ),
    mesh=_mesh,
    scratch_types=(
        [pltpu.VMEM((128,), jnp.int32) for _ in range(5)]
        + [pltpu.VMEM((128, 64), jnp.float32) for _ in range(2)]
        + [pltpu.VMEM((16,), jnp.int32),
           pltpu.VMEM((128, 64), jnp.float32)]
        + [pltpu.VMEM_SHARED((NACC, 64), jnp.float32)]
        + [pltpu.SemaphoreType.DMA for _ in range(6)]
    ),
    compiler_params=pltpu.CompilerParams(use_tc_tiling_on_sc=False,
                                         needs_layout_passes=False),
)


def _dinv_body(degp_ref, dinv_ref):
    deg = jnp.sum(degp_ref[...], axis=0)
    dinv_ref[...] = jnp.where(deg > 0, lax.rsqrt(jnp.maximum(deg, 1e-12)), 0.0)


def _tc_dinv(degp):
    return pl.pallas_call(
        _dinv_body,
        out_shape=jax.ShapeDtypeStruct((3, NP), jnp.float32),
    )(degp)


def _scale_body(x_ref, dinv_ref, hb_ref):
    x = x_ref[...]
    for b in range(3):
        db = dinv_ref[:, b][:, None]
        hb_ref[b, 0] = db * x[:, :64]
        hb_ref[b, 1] = db * x[:, 64:]


def _tc_scale(x, dinv):
    blk = 1000
    grid = N // blk
    return pl.pallas_call(
        _scale_body,
        grid=(grid,),
        in_specs=[
            pl.BlockSpec((blk, D), lambda i: (i, 0)),
            pl.BlockSpec((blk, 3), lambda i: (i, 0)),
        ],
        out_specs=pl.BlockSpec((3, 2, blk, 64), lambda i: (0, 0, i, 0)),
        out_shape=jax.ShapeDtypeStruct((3, 2, N, 64), jnp.float32),
    )(x, dinv)


def _layer_a_body(h_ref, s0_ref, s1_ref, s2_ref, dinv_ref,
                  w_ref, bias_ref, u_ref, stats_ref):
    srefs = [s0_ref, s1_ref, s2_ref]
    acc = None
    for b in range(3):
        sb = jnp.concatenate([srefs[b][0], srefs[b][1]], axis=1)
        pre = dinv_ref[:, b][:, None] * sb
        ob = jnp.maximum(
            jnp.dot(pre, w_ref[b], preferred_element_type=jnp.float32)
            + bias_ref[b, :][None, :], 0.0)
        acc = ob if acc is None else acc + ob
    xi = jnp.maximum(
        jnp.dot(h_ref[...], w_ref[3], preferred_element_type=jnp.float32)
        + bias_ref[3, :][None, :], 0.0)
    u = acc + xi
    u_ref[...] = u

    @pl.when(pl.program_id(0) == 0)
    def _():
        stats_ref[...] = jnp.zeros_like(stats_ref)

    stats_ref[0:1, :] += jnp.sum(u, axis=0, keepdims=True)
    stats_ref[1:2, :] += jnp.sum(u * u, axis=0, keepdims=True)


def _tc_layer_a(h, s0, s1, s2, dinv, w4, b4):
    blk = 1000
    grid = N // blk
    sspec = pl.BlockSpec((2, blk, 64), lambda i: (0, i, 0))
    return pl.pallas_call(
        _layer_a_body,
        grid=(grid,),
        in_specs=[
            pl.BlockSpec((blk, D), lambda i: (i, 0)),
            sspec, sspec, sspec,
            pl.BlockSpec((blk, 3), lambda i: (i, 0)),
            pl.BlockSpec((4, D, H), lambda i: (0, 0, 0)),
            pl.BlockSpec((4, H), lambda i: (0, 0)),
        ],
        out_specs=(
            pl.BlockSpec((blk, H), lambda i: (i, 0)),
            pl.BlockSpec((2, H), lambda i: (0, 0)),
        ),
        out_shape=(
            jax.ShapeDtypeStruct((N, H), jnp.float32),
            jax.ShapeDtypeStruct((2, H), jnp.float32),
        ),
    )(h, s0, s1, s2, dinv, w4, b4)


def _layer_b_body(u_ref, stats_ref, gb_ref, dinv_ref, h_ref,
                  hb_ref, *, make_hb):
    mean = stats_ref[0:1, :] / N
    var = stats_ref[1:2, :] / N - mean * mean
    rstd = lax.rsqrt(var + 1e-5)
    hn = (u_ref[...] - mean) * rstd * gb_ref[0:1, :] + gb_ref[1:2, :]
    h_ref[...] = hn
    if make_hb:
        for b in range(3):
            db = dinv_ref[:, b][:, None]
            hb_ref[b, 0] = db * hn[:, :64]
            hb_ref[b, 1] = db * hn[:, 64:]


def _tc_layer_b(u, stats, gb, dinv, make_hb):
    blk = 1000
    grid = N // blk
    body = functools.partial(_layer_b_body, make_hb=make_hb)
    if make_hb:
        out_specs = (
            pl.BlockSpec((blk, H), lambda i: (i, 0)),
            pl.BlockSpec((3, 2, blk, 64), lambda i: (0, 0, i, 0)),
        )
        out_shape = (
            jax.ShapeDtypeStruct((N, H), jnp.float32),
            jax.ShapeDtypeStruct((3, 2, N, 64), jnp.float32),
        )
    else:
        def body(u_ref, stats_ref, gb_ref, dinv_ref, h_ref):
            _layer_b_body(u_ref, stats_ref, gb_ref, dinv_ref, h_ref,
                          None, make_hb=False)
        out_specs = pl.BlockSpec((blk, H), lambda i: (i, 0))
        out_shape = jax.ShapeDtypeStruct((N, H), jnp.float32)
    return pl.pallas_call(
        body,
        grid=(grid,),
        in_specs=[
            pl.BlockSpec((blk, H), lambda i: (i, 0)),
            pl.BlockSpec((2, H), lambda i: (0, 0)),
            pl.BlockSpec((2, H), lambda i: (0, 0)),
            pl.BlockSpec((blk, 3), lambda i: (i, 0)),
        ],
        out_specs=out_specs,
        out_shape=out_shape,
    )(u, stats, gb, dinv)


def _pool_body(h_ref, batch_ref, pooled_ref, counts_ref):
    bt = batch_ref[0, 0, :]
    gi = lax.broadcasted_iota(jnp.int32, (G, bt.shape[0]), 0)
    oh = (gi == bt[None, :]).astype(jnp.float32)

    @pl.when(pl.program_id(0) == 0)
    def _():
        pooled_ref[...] = jnp.zeros_like(pooled_ref)
        counts_ref[...] = jnp.zeros_like(counts_ref)

    pooled_ref[...] += jnp.dot(oh, h_ref[...],
                               preferred_element_type=jnp.float32)
    counts_ref[...] += jnp.sum(oh, axis=1)[None, :]


def _tc_pool(h, batch3d):
    blk = 1000
    grid = N // blk
    return pl.pallas_call(
        _pool_body,
        grid=(grid,),
        in_specs=[
            pl.BlockSpec((blk, H), lambda i: (i, 0)),
            pl.BlockSpec((1, 1, blk), lambda i: (i, 0, 0)),
        ],
        out_specs=(
            pl.BlockSpec((G, H), lambda i: (0, 0)),
            pl.BlockSpec((1, G), lambda i: (0, 0)),
        ),
        out_shape=(
            jax.ShapeDtypeStruct((G, H), jnp.float32),
            jax.ShapeDtypeStruct((1, G), jnp.float32),
        ),
    )(h, batch3d)


def _mlp_body(pooled_ref, counts_ref, w1a_ref, w1b_ref, b1_ref,
              w2_ref, b2_ref, out_ref):
    cnt = counts_ref[0, :][:, None] / 40.0
    z = (jnp.dot(pooled_ref[...], w1a_ref[...],
                 preferred_element_type=jnp.float32)
         + cnt * w1b_ref[0:1, :] + b1_ref[0:1, :])
    z = jnp.maximum(z, 0.0)
    out_ref[...] = (jnp.dot(z, w2_ref[...],
                            preferred_element_type=jnp.float32)
                    + b2_ref[0:1, :])


def _tc_mlp(pooled, counts, w1a, w1b, b1, w2, b2):
    return pl.pallas_call(
        _mlp_body,
        out_shape=jax.ShapeDtypeStruct((G, C), jnp.float32),
    )(pooled, counts, w1a, w1b, b1, w2, b2)


def kernel(x, edge_attr, params, edge_index, batch):
    row = edge_index[0]
    col = edge_index[1]
    ewT = jnp.transpose(edge_attr[:, :3]).reshape(-1)

    degp, colc, rowc, cnt = _sc_prep(col, row, ewT)
    dinv_p = _tc_dinv(degp.reshape(NW, 3, NP))
    dinv = jnp.transpose(dinv_p)

    hb = _tc_scale(x, dinv).reshape(6 * N, 64)

    h = x
    for li, lyr in enumerate(params["layers"]):
        s0, s1, s2 = _sc_agg(hb, rowc, colc, cnt)
        w4 = jnp.stack([lyr["Ws"], lyr["Wd"], lyr["Wt"], lyr["Wi"]])
        b4 = jnp.stack([lyr["bs"], lyr["bd"], lyr["bt"], lyr["bi"]])
        u, stats = _tc_layer_a(h, s0, s1, s2, dinv, w4, b4)
        gb = jnp.stack([lyr["g"], lyr["be"]])
        if li == 0:
            h, hb4 = _tc_layer_b(u, stats, gb, dinv, True)
            hb = hb4.reshape(6 * N, 64)
        else:
            h = _tc_layer_b(u, stats, gb, dinv, False)

    batch3d = batch.reshape(10, 1, N // 10)
    pooled, counts = _tc_pool(h, batch3d)
    w1a = params["fc1_W"][:H, :]
    w1b = params["fc1_W"][H:, :]
    return _tc_mlp(pooled, counts, w1a, w1b,
                   params["fc1_b"][None, :], params["fc2_W"],
                   params["fc2_b"][None, :])
